# initial kernel scaffold (unmeasured)
import jax
import jax.numpy as jnp
from jax import lax
from jax.experimental import pallas as pl
from jax.experimental.pallas import tpu as pltpu


def kernel(
    x,
):
    def body(*refs):
        pass

    out_shape = jax.ShapeDtypeStruct(..., jnp.float32)
    return pl.pallas_call(body, out_shape=out_shape)(...)


# baseline (device time: 46361 ns/iter reference)
import jax
import jax.numpy as jnp
from jax import lax
from jax.experimental import pallas as pl
from jax.experimental.pallas import tpu as pltpu

N_Z = 4


def kernel(x):
    m_per, n = x.shape

    def body(x_ref, out_ref, comm_ref, send_sems, recv_sems):
        my_x = lax.axis_index("x")
        my_y = lax.axis_index("y")
        my_z = lax.axis_index("z")
        left = (my_z - 1) % N_Z
        right = (my_z + 1) % N_Z

        barrier_sem = pltpu.get_barrier_semaphore()
        for nbr in [left, right]:
            pl.semaphore_signal(
                barrier_sem,
                inc=1,
                device_id=(my_x, my_y, nbr),
                device_id_type=pl.DeviceIdType.MESH,
            )
        pl.semaphore_wait(barrier_sem, 2)

        chunk = x_ref[:, :].astype(jnp.bfloat16)
        out_ref[pl.ds(my_z * m_per, m_per), :] = chunk
        comm_ref[0, :, :] = chunk

        for h in range(N_Z - 1):
            send_slot = h % 2
            recv_slot = (h + 1) % 2
            rdma = pltpu.make_async_remote_copy(
                src_ref=comm_ref.at[send_slot],
                dst_ref=comm_ref.at[recv_slot],
                send_sem=send_sems.at[send_slot],
                recv_sem=recv_sems.at[recv_slot],
                device_id=(my_x, my_y, right),
                device_id_type=pl.DeviceIdType.MESH,
            )
            rdma.start()
            rdma.wait()

            origin = (my_z - h - 1) % N_Z
            out_ref[pl.ds(origin * m_per, m_per), :] = comm_ref[recv_slot, :, :]

    return pl.pallas_call(
        body,
        out_shape=jax.ShapeDtypeStruct((N_Z * m_per, n), jnp.bfloat16),
        in_specs=[pl.BlockSpec(memory_space=pltpu.VMEM)],
        out_specs=pl.BlockSpec(memory_space=pltpu.VMEM),
        scratch_shapes=[
            pltpu.VMEM((2, m_per, n), jnp.bfloat16),
            pltpu.SemaphoreType.DMA((2,)),
            pltpu.SemaphoreType.DMA((2,)),
        ],
        compiler_params=pltpu.CompilerParams(collective_id=0),
    )(x)


# device time: 35307 ns/iter; 1.3131x vs baseline; 1.3131x over previous
import jax
import jax.numpy as jnp
from jax import lax
from jax.experimental import pallas as pl
from jax.experimental.pallas import tpu as pltpu

N_Z = 4


def kernel(x):
    m_per, n = x.shape
    q = m_per // 4

    def body(x_ref, out_ref, zsend, zrecv, xsend, xrecv, ysend, yrecv,
             dsend, drecv):
        mx = lax.axis_index("x")
        my = lax.axis_index("y")
        mz = lax.axis_index("z")
        qoff = (2 * mx + my) * q

        def rows(chunk, off):
            return pl.ds(chunk * m_per + off, q)

        def copy(r, send_sem, recv_sem, dev):
            return pltpu.make_async_remote_copy(
                src_ref=out_ref.at[r, :],
                dst_ref=out_ref.at[r, :],
                send_sem=send_sem,
                recv_sem=recv_sem,
                device_id=dev,
                device_id_type=pl.DeviceIdType.MESH,
            )

        bar = pltpu.get_barrier_semaphore()
        for s in range(N_Z):
            @pl.when(s != mz)
            def _():
                pl.semaphore_signal(
                    bar, inc=1, device_id=(mx, my, s),
                    device_id_type=pl.DeviceIdType.MESH)
        for dev in ((1 - mx, my, mz), (mx, 1 - my, mz), (1 - mx, 1 - my, mz)):
            pl.semaphore_signal(
                bar, inc=1, device_id=dev,
                device_id_type=pl.DeviceIdType.MESH)
        pl.semaphore_wait(bar, 6)

        out_ref[pl.ds(mz * m_per, m_per), :] = x_ref[:, :].astype(jnp.bfloat16)

        for s in range(N_Z):
            @pl.when(s != mz)
            def _():
                copy(rows(mz, qoff), zsend.at[s], zrecv.at[mz],
                     (mx, my, s)).start()

        for d in (1, 2, 3):
            for sg in (-1, 1):
                s_val = mz + sg * d
                @pl.when((s_val >= 0) & (s_val < N_Z))
                def _():
                    r = rows(s_val, qoff)
                    copy(r, zsend.at[s_val], zrecv.at[s_val],
                         (mx, my, mz)).wait_recv()
                    copy(r, xsend.at[s_val], xrecv.at[s_val],
                         (1 - mx, my, mz)).start()
                    copy(r, ysend.at[s_val], yrecv.at[s_val],
                         (mx, 1 - my, mz)).start()
                    copy(r, dsend.at[s_val], drecv.at[s_val],
                         (1 - mx, 1 - my, mz)).start()

        for s in range(N_Z):
            @pl.when(s != mz)
            def _():
                rx = rows(s, (2 * (1 - mx) + my) * q)
                copy(rx, xsend.at[s], xrecv.at[s], (mx, my, mz)).wait_recv()
                ry = rows(s, (2 * mx + (1 - my)) * q)
                copy(ry, ysend.at[s], yrecv.at[s], (mx, my, mz)).wait_recv()
                rd = rows(s, (2 * (1 - mx) + (1 - my)) * q)
                copy(rd, dsend.at[s], drecv.at[s], (mx, my, mz)).wait_recv()

        for s in range(N_Z):
            @pl.when(s != mz)
            def _():
                copy(rows(mz, qoff), zsend.at[s], zrecv.at[mz],
                     (mx, my, mz)).wait_send()
                r = rows(s, qoff)
                copy(r, xsend.at[s], xrecv.at[s], (mx, my, mz)).wait_send()
                copy(r, ysend.at[s], yrecv.at[s], (mx, my, mz)).wait_send()
                copy(r, dsend.at[s], drecv.at[s], (mx, my, mz)).wait_send()

    return pl.pallas_call(
        body,
        out_shape=jax.ShapeDtypeStruct((N_Z * m_per, n), jnp.bfloat16),
        in_specs=[pl.BlockSpec(memory_space=pltpu.VMEM)],
        out_specs=pl.BlockSpec(memory_space=pltpu.VMEM),
        scratch_shapes=[
            pltpu.SemaphoreType.DMA((N_Z,)),
            pltpu.SemaphoreType.DMA((N_Z,)),
            pltpu.SemaphoreType.DMA((N_Z,)),
            pltpu.SemaphoreType.DMA((N_Z,)),
            pltpu.SemaphoreType.DMA((N_Z,)),
            pltpu.SemaphoreType.DMA((N_Z,)),
            pltpu.SemaphoreType.DMA((N_Z,)),
            pltpu.SemaphoreType.DMA((N_Z,)),
        ],
        compiler_params=pltpu.CompilerParams(collective_id=0),
    )(x)


# device time: 35252 ns/iter; 1.3151x vs baseline; 1.0016x over previous
import jax
import jax.numpy as jnp
from jax import lax
from jax.experimental import pallas as pl
from jax.experimental.pallas import tpu as pltpu

N_Z = 4


def kernel(x):
    m_per, n = x.shape
    q = m_per // 4

    def body(x_ref, out_ref, zsend, zrecv, xsend, xrecv, ysend, yrecv,
             dsend, drecv):
        mx = lax.axis_index("x")
        my = lax.axis_index("y")
        mz = lax.axis_index("z")
        qoff = (2 * mx + my) * q

        def rows(chunk, off):
            return pl.ds(chunk * m_per + off, q)

        def copy(r, send_sem, recv_sem, dev):
            return pltpu.make_async_remote_copy(
                src_ref=out_ref.at[r, :],
                dst_ref=out_ref.at[r, :],
                send_sem=send_sem,
                recv_sem=recv_sem,
                device_id=dev,
                device_id_type=pl.DeviceIdType.MESH,
            )

        out_ref[pl.ds(mz * m_per + qoff, q), :] = (
            x_ref[pl.ds(qoff, q), :].astype(jnp.bfloat16))

        bar = pltpu.get_barrier_semaphore()
        for s in range(N_Z):
            @pl.when(s != mz)
            def _():
                pl.semaphore_signal(
                    bar, inc=1, device_id=(mx, my, s),
                    device_id_type=pl.DeviceIdType.MESH)
        for dev in ((1 - mx, my, mz), (mx, 1 - my, mz), (1 - mx, 1 - my, mz)):
            pl.semaphore_signal(
                bar, inc=1, device_id=dev,
                device_id_type=pl.DeviceIdType.MESH)
        pl.semaphore_wait(bar, 6)

        for s in range(N_Z):
            @pl.when(s != mz)
            def _():
                copy(rows(mz, qoff), zsend.at[s], zrecv.at[mz],
                     (mx, my, s)).start()

        qidx = 2 * mx + my
        for k in range(4):
            @pl.when(qidx != k)
            def _(k=k):
                out_ref[pl.ds(mz * m_per + k * q, q), :] = (
                    x_ref[k * q:(k + 1) * q, :].astype(jnp.bfloat16))

        for d in (1, 2, 3):
            for sg in (-1, 1):
                s_val = mz + sg * d
                @pl.when((s_val >= 0) & (s_val < N_Z))
                def _():
                    r = rows(s_val, qoff)
                    copy(r, zsend.at[s_val], zrecv.at[s_val],
                         (mx, my, mz)).wait_recv()
                    copy(r, xsend.at[s_val], xrecv.at[s_val],
                         (1 - mx, my, mz)).start()
                    copy(r, ysend.at[s_val], yrecv.at[s_val],
                         (mx, 1 - my, mz)).start()
                    copy(r, dsend.at[s_val], drecv.at[s_val],
                         (1 - mx, 1 - my, mz)).start()

        for s in range(N_Z):
            @pl.when(s != mz)
            def _():
                rx = rows(s, (2 * (1 - mx) + my) * q)
                copy(rx, xsend.at[s], xrecv.at[s], (mx, my, mz)).wait_recv()
                ry = rows(s, (2 * mx + (1 - my)) * q)
                copy(ry, ysend.at[s], yrecv.at[s], (mx, my, mz)).wait_recv()
                rd = rows(s, (2 * (1 - mx) + (1 - my)) * q)
                copy(rd, dsend.at[s], drecv.at[s], (mx, my, mz)).wait_recv()

        for s in range(N_Z):
            @pl.when(s != mz)
            def _():
                copy(rows(mz, qoff), zsend.at[s], zrecv.at[mz],
                     (mx, my, mz)).wait_send()
                r = rows(s, qoff)
                copy(r, xsend.at[s], xrecv.at[s], (mx, my, mz)).wait_send()
                copy(r, ysend.at[s], yrecv.at[s], (mx, my, mz)).wait_send()
                copy(r, dsend.at[s], drecv.at[s], (mx, my, mz)).wait_send()

    return pl.pallas_call(
        body,
        out_shape=jax.ShapeDtypeStruct((N_Z * m_per, n), jnp.bfloat16),
        in_specs=[pl.BlockSpec(memory_space=pltpu.VMEM)],
        out_specs=pl.BlockSpec(memory_space=pltpu.VMEM),
        scratch_shapes=[
            pltpu.SemaphoreType.DMA((N_Z,)),
            pltpu.SemaphoreType.DMA((N_Z,)),
            pltpu.SemaphoreType.DMA((N_Z,)),
            pltpu.SemaphoreType.DMA((N_Z,)),
            pltpu.SemaphoreType.DMA((N_Z,)),
            pltpu.SemaphoreType.DMA((N_Z,)),
            pltpu.SemaphoreType.DMA((N_Z,)),
            pltpu.SemaphoreType.DMA((N_Z,)),
        ],
        compiler_params=pltpu.CompilerParams(collective_id=0),
    )(x)
